# baseline (speedup 1.0000x reference)
import functools

import jax
import jax.numpy as jnp
from jax import lax
from jax.experimental import pallas as pl
from jax.experimental.pallas import tpu as pltpu
from jax.experimental.pallas import tpu_sc as plsc

_B = 128
_SB = 2 * _B
_BB = 8
_G = _SB // _BB
_EMB = 128
_MID = 256
_E = 8
_NC = 10
_GROUPS = 8
_EPS = 1e-5
_SQRT2 = 1.4142135623730951
_PREC = lax.Precision.DEFAULT
_PREC_STAT = lax.Precision.HIGHEST


def _gelu(x):
    return x * 0.5 * (1.0 + lax.erf(x / _SQRT2))


def _group_stats(y, groups):
    t, c = y.shape
    cpg = c // groups
    n = float(t * cpg)
    s1 = jnp.sum(y, axis=0, keepdims=True)
    s2 = jnp.sum(y * y, axis=0, keepdims=True)
    gi = lax.broadcasted_iota(jnp.int32, (c, groups), 0) // cpg
    gj = lax.broadcasted_iota(jnp.int32, (c, groups), 1)
    m = (gi == gj).astype(jnp.float32)
    ti = lax.broadcasted_iota(jnp.int32, (groups, c), 0)
    tj = lax.broadcasted_iota(jnp.int32, (groups, c), 1) // cpg
    mt = (ti == tj).astype(jnp.float32)
    gmean = jnp.dot(s1, m, preferred_element_type=jnp.float32, precision=_PREC_STAT) / n
    gsq = jnp.dot(s2, m, preferred_element_type=jnp.float32, precision=_PREC_STAT) / n
    gvar = gsq - gmean * gmean
    grstd = lax.rsqrt(gvar + _EPS)
    mean_c = jnp.dot(gmean, mt, preferred_element_type=jnp.float32, precision=_PREC_STAT)
    rstd_c = jnp.dot(grstd, mt, preferred_element_type=jnp.float32, precision=_PREC_STAT)
    return mean_c, rstd_c


def _gn_gelu(y, gs, gb, groups):
    mean_c, rstd_c = _group_stats(y, groups)
    yn = (y - mean_c) * rstd_c * gs + gb
    return _gelu(yn)


def _c1_body(x_ref, w_ref, r_ref, gs_ref, gb_ref, o_ref):
    w = w_ref[0]
    r = r_ref[0]
    gs = gs_ref[0]
    gb = gb_ref[0]
    for s in range(_BB):
        sm2 = x_ref[s, 2:258, 16:32]
        sm1 = x_ref[s, 3:259, :]
        s0 = x_ref[s, 4:260, :]
        sp1 = x_ref[s, 5:261, :]
        xcat = jnp.concatenate([sm2, sm1, s0, sp1], axis=1)
        y = jnp.dot(xcat, w, preferred_element_type=jnp.float32, precision=_PREC)
        res = jnp.dot(s0[:, 0:16], r, preferred_element_type=jnp.float32, precision=_PREC)
        o_ref[s] = _gn_gelu(y, gs, gb, _GROUPS) + res


def _c2_body(x_ref, wm1_ref, w0_ref, wp1_ref, gs_ref, gb_ref, o_ref):
    wm1 = wm1_ref[0]
    w0 = w0_ref[0]
    wp1 = wp1_ref[0]
    gs = gs_ref[0]
    gb = gb_ref[0]
    for s in range(_BB):
        sm1 = x_ref[s, 3:131, :]
        s0 = x_ref[s, 4:132, :]
        sp1 = x_ref[s, 5:133, 0:128]
        acc0 = jnp.dot(s0, w0, preferred_element_type=jnp.float32, precision=_PREC)
        y = (jnp.dot(sm1, wm1, preferred_element_type=jnp.float32, precision=_PREC)
             + acc0[:, 0:256]
             + jnp.dot(sp1, wp1, preferred_element_type=jnp.float32, precision=_PREC))
        res = acc0[:, 256:512]
        o_ref[s] = _gn_gelu(y, gs, gb, _GROUPS) + res


def _c3_body(x_ref, wm1_ref, w0_ref, wp1_ref, gs_ref, gb_ref, ts_ref, tb_ref,
             o_ref):
    wm1 = wm1_ref[0]
    w0 = w0_ref[0]
    wp1 = wp1_ref[0]
    gs = gs_ref[0]
    gb = gb_ref[0]
    ts = ts_ref[0]
    tb = tb_ref[0]
    rows = []
    for s in range(_BB):
        sm1 = x_ref[s, 3:131, :]
        s0 = x_ref[s, 4:132, :]
        sp1 = x_ref[s, 5:133, :]
        acc0 = jnp.dot(s0, w0, preferred_element_type=jnp.float32, precision=_PREC)
        y = (jnp.dot(sm1, wm1, preferred_element_type=jnp.float32, precision=_PREC)
             + acc0[:, 0:128]
             + jnp.dot(sp1, wp1, preferred_element_type=jnp.float32, precision=_PREC))
        z = _gn_gelu(y, gs, gb, _GROUPS) + acc0[:, 128:256]
        mean_c, rstd_c = _group_stats(z, _GROUPS)
        z = (z - mean_c) * rstd_c * ts + tb
        rows.append(jnp.mean(z, axis=0, keepdims=True))
    o_ref[...] = jnp.concatenate(rows, axis=0)


def _head1_body(he_ref, hi_ref, a_ref, b_ref, fb_ref, gw_ref, gb_ref,
                z_ref, gl_ref):
    t = (jnp.dot(he_ref[...], a_ref[...], preferred_element_type=jnp.float32, precision=_PREC)
         + jnp.dot(hi_ref[...], b_ref[...], preferred_element_type=jnp.float32, precision=_PREC)
         + fb_ref[...])
    z = _gelu(t)
    z_ref[...] = z
    gl_ref[...] = (jnp.dot(z, gw_ref[...], preferred_element_type=jnp.float32, precision=_PREC)
                   + gb_ref[...])


def _head2_body(z_ref, g_ref, w1_ref, b1_ref, lns_ref, lnb_ref, w2_ref, b2_ref,
                o_ref):
    z = z_ref[...]
    g = g_ref[...]
    acc = jnp.zeros((_B, _NC), jnp.float32)
    for e in range(_E):
        zz = jnp.dot(z, w1_ref[e], preferred_element_type=jnp.float32, precision=_PREC) + b1_ref[e]
        zz = _gelu(zz)
        mu = jnp.mean(zz, axis=1, keepdims=True)
        var = jnp.mean(zz * zz, axis=1, keepdims=True) - mu * mu
        zz = (zz - mu) * lax.rsqrt(var + _EPS) * lns_ref[e] + lnb_ref[e]
        le = jnp.dot(zz, w2_ref[e], preferred_element_type=jnp.float32, precision=_PREC) + b2_ref[e]
        acc = acc + g[:, e:e + 1] * le
    o_ref[...] = acc


def _sc_gates(gl_flat):
    mesh = plsc.VectorSubcoreMesh(core_axis_name="c", subcore_axis_name="s")
    chunk = _E * 16

    @functools.partial(
        pl.kernel, mesh=mesh,
        out_type=jax.ShapeDtypeStruct((_B * _E,), jnp.float32),
        scratch_types=[pltpu.VMEM((chunk,), jnp.float32),
                       pltpu.VMEM((chunk,), jnp.float32),
                       pltpu.SemaphoreType.DMA],
    )
    def k(gl_hbm, out_hbm, in_v, out_v, sem):
        wid = lax.axis_index("s") * 2 + lax.axis_index("c")

        @pl.when(wid < _B // 16)
        def _():
            base = wid * chunk
            pltpu.sync_copy(gl_hbm.at[pl.ds(base, chunk)], in_v)
            m1 = in_v[pl.ds(0, 16)]
            i1 = jnp.zeros((16,), jnp.int32)
            for e in range(1, _E):
                ve = in_v[pl.ds(e * 16, 16)]
                gt = ve > m1
                m1 = jnp.where(gt, ve, m1)
                i1 = jnp.where(gt, jnp.full((16,), e, jnp.int32), i1)
            m2 = jnp.full((16,), -3.0e38, jnp.float32)
            i2 = jnp.zeros((16,), jnp.int32)
            for e in range(_E):
                ve = in_v[pl.ds(e * 16, 16)]
                ok = jnp.logical_and(i1 != e, ve > m2)
                m2 = jnp.where(ok, ve, m2)
                i2 = jnp.where(ok, jnp.full((16,), e, jnp.int32), i2)
            t = jnp.exp(m2 - m1)
            denom = 1.0 + t
            p1 = 1.0 / denom
            p2 = t / denom
            zero = jnp.zeros((16,), jnp.float32)
            for e in range(_E):
                out_v[pl.ds(e * 16, 16)] = (jnp.where(i1 == e, p1, zero)
                                            + jnp.where(i2 == e, p2, zero))
            pltpu.sync_copy(out_v, out_hbm.at[pl.ds(base, chunk)])

    return k(gl_flat)


def _conv_call(body, x, weights, t_out, c_out, interpret=False):
    n_t = x.shape[1]
    c_in = x.shape[2]
    in_specs = [pl.BlockSpec((_BB, n_t, c_in), lambda g: (g, 0, 0))]
    for wa in weights:
        shp = wa.shape
        in_specs.append(pl.BlockSpec((1,) + shp[1:],
                                     lambda g: (g // (_G // 2), 0, 0)))
    if t_out == 1:
        out_spec = pl.BlockSpec((_BB, c_out), lambda g: (g, 0))
        out_shape = jax.ShapeDtypeStruct((_SB, c_out), jnp.float32)
    else:
        out_spec = pl.BlockSpec((_BB, t_out, c_out), lambda g: (g, 0, 0))
        out_shape = jax.ShapeDtypeStruct((_SB, t_out, c_out), jnp.float32)
    return pl.pallas_call(
        body,
        grid=(_G,),
        in_specs=in_specs,
        out_specs=out_spec,
        out_shape=out_shape,
        interpret=interpret,
    )(x, *weights)


def _prep_encoder_weights(p):
    def stack(f):
        return jnp.stack([f('emg'), f('imu')], axis=0)

    def w1m(pre):
        w = p[pre + '_w1']
        if w.shape[1] < 16:
            w = jnp.pad(w, ((0, 0), (0, 16 - w.shape[1]), (0, 0)))
        return jnp.transpose(w, (2, 1, 0)).reshape(112, 128)

    def r1m(pre):
        r = p[pre + '_r1'][:, :, 0]
        if r.shape[1] < 16:
            r = jnp.pad(r, ((0, 0), (0, 16 - r.shape[1])))
        return r.T

    def tap(pre, name, d):
        return p[pre + name][:, :, d].T

    def w2m1(pre):
        return jnp.concatenate([tap(pre, '_w2', 0), tap(pre, '_w2', 1)], axis=0)

    def w20(pre):
        center = jnp.concatenate([tap(pre, '_w2', 2), tap(pre, '_w2', 3)], axis=0)
        resid = jnp.concatenate([p[pre + '_r2'][:, :, 0].T,
                                 jnp.zeros((128, 256), jnp.float32)], axis=0)
        return jnp.concatenate([center, resid], axis=1)

    def w30(pre):
        return jnp.concatenate([tap(pre, '_w3', 1), p[pre + '_r3'][:, :, 0].T],
                               axis=1)

    def gn(name):
        return stack(lambda pre: p[pre + name][None, :])

    return {
        'w1': stack(w1m), 'r1': stack(r1m),
        'g1s': gn('_g1s'), 'g1b': gn('_g1b'),
        'w2m1': stack(w2m1), 'w20': stack(w20),
        'w2p1': stack(lambda pre: tap(pre, '_w2', 4)),
        'g2s': gn('_g2s'), 'g2b': gn('_g2b'),
        'w3m1': stack(lambda pre: tap(pre, '_w3', 0)),
        'w30': stack(w30),
        'w3p1': stack(lambda pre: tap(pre, '_w3', 2)),
        'g3s': gn('_g3s'), 'g3b': gn('_g3b'),
        'gts': gn('_gts'), 'gtb': gn('_gtb'),
    }


def _forward(emg, imu, p, interpret=False, gates_fn=None):
    w = _prep_encoder_weights(p)

    emg_t = jnp.transpose(emg, (0, 2, 1))
    imu_t = jnp.pad(jnp.transpose(imu, (0, 2, 1)),
                    ((0, 0), (0, 0), (0, 10)))
    x = jnp.concatenate([emg_t, imu_t], axis=0)
    x1 = jnp.pad(x.reshape(_SB, 256, 32), ((0, 0), (4, 4), (0, 0)))

    y1 = _conv_call(_c1_body, x1,
                    [w['w1'], w['r1'], w['g1s'], w['g1b']],
                    256, 128, interpret)
    x2 = jnp.pad(y1.reshape(_SB, 128, 256), ((0, 0), (4, 4), (0, 0)))

    y2 = _conv_call(_c2_body, x2,
                    [w['w2m1'], w['w20'], w['w2p1'], w['g2s'], w['g2b']],
                    128, 256, interpret)
    x3 = jnp.pad(y2, ((0, 0), (4, 4), (0, 0)))

    h = _conv_call(_c3_body, x3,
                   [w['w3m1'], w['w30'], w['w3p1'], w['g3s'], w['g3b'],
                    w['gts'], w['gtb']],
                   1, 128, interpret)

    h_e = h[:_B]
    h_i = h[_B:]
    a = p['fuse_w'][:, :128].T
    b = p['fuse_w'][:, 128:].T
    fb = p['fuse_b'][None, :]
    gw = p['gate_w'].T
    gb = p['gate_b'][None, :]

    z, gl = pl.pallas_call(
        _head1_body,
        out_shape=[jax.ShapeDtypeStruct((_B, _EMB), jnp.float32),
                   jax.ShapeDtypeStruct((_B, _E), jnp.float32)],
        interpret=interpret,
    )(h_e, h_i, a, b, fb, gw, gb)

    if gates_fn is None:
        gl_flat = jnp.transpose(gl.reshape(_B // 16, 16, _E),
                                (0, 2, 1)).reshape(_B * _E)
        gates = jnp.transpose(_sc_gates(gl_flat).reshape(_B // 16, _E, 16),
                              (0, 2, 1)).reshape(_B, _E)
    else:
        gates = gates_fn(gl)

    out = pl.pallas_call(
        _head2_body,
        out_shape=jax.ShapeDtypeStruct((_B, _NC), jnp.float32),
        interpret=interpret,
    )(z, gates, p['exp_w1'], p['exp_b1'][:, None, :],
      p['exp_lns'][:, None, :], p['exp_lnb'][:, None, :],
      p['exp_w2'], p['exp_b2'][:, None, :])
    return out


def kernel(emg, imu, params):
    return _forward(emg, imu, params)


# speedup vs baseline: 2.0055x; 2.0055x over previous
import functools

import jax
import jax.numpy as jnp
from jax import lax
from jax.experimental import pallas as pl
from jax.experimental.pallas import tpu as pltpu
from jax.experimental.pallas import tpu_sc as plsc

_B = 128
_SB = 2 * _B
_BB = 8
_G = _SB // _BB
_EMB = 128
_MID = 256
_E = 8
_NC = 10
_GROUPS = 8
_EPS = 1e-5
_SQRT2 = 1.4142135623730951
_PREC = lax.Precision.DEFAULT
_PREC_STAT = lax.Precision.HIGHEST

_SREG = 144
_MFULL = _BB * _SREG
_MV = _MFULL - 16


def _gelu(x):
    return x * 0.5 * (1.0 + lax.erf(x / _SQRT2))


def _dotd(a, b):
    return jnp.dot(a, b, preferred_element_type=jnp.float32, precision=_PREC)


def _dots(a, b):
    return jnp.dot(a, b, preferred_element_type=jnp.float32,
                   precision=_PREC_STAT)


def _msel():
    r = lax.broadcasted_iota(jnp.int32, (_BB, _MV), 1)
    s = lax.broadcasted_iota(jnp.int32, (_BB, _MV), 0)
    return ((r // _SREG == s) & (r % _SREG < 128)).astype(jnp.float32)


def _maskcol():
    r = lax.broadcasted_iota(jnp.int32, (_MV, 1), 0)
    return (r % _SREG < 128).astype(jnp.float32)


def _groupmats(c, cpg):
    gi = lax.broadcasted_iota(jnp.int32, (c, _GROUPS), 0) // cpg
    gj = lax.broadcasted_iota(jnp.int32, (c, _GROUPS), 1)
    mg = (gi == gj).astype(jnp.float32)
    ti = lax.broadcasted_iota(jnp.int32, (_GROUPS, c), 0)
    tj = lax.broadcasted_iota(jnp.int32, (_GROUPS, c), 1) // cpg
    mgt = (ti == tj).astype(jnp.float32)
    return mg, mgt


def _expand(a):
    c = a.shape[1]
    full = jnp.broadcast_to(a[:, None, :], (_BB, _SREG, c)).reshape(_MFULL, c)
    return full[0:_MV]


def _ab_maps(s1, s2, mg, mgt, gs, gb, n):
    g1 = _dots(s1, mg) / n
    g2 = _dots(s2, mg) / n
    grstd = lax.rsqrt(g2 - g1 * g1 + _EPS)
    mean_c = _dots(g1, mgt)
    rstd_c = _dots(grstd, mgt)
    a = rstd_c * gs
    b = gb - mean_c * a
    return a, b


def _enc_body(x_ref, w1m1_ref, w10_ref, w1p1_ref, g1s_ref, g1b_ref,
              w2m1_ref, w20_ref, w2p1_ref, g2s_ref, g2b_ref,
              w3m1_ref, w30_ref, w3p1_ref, g3s_ref, g3b_ref,
              ts_ref, tb_ref, o_ref, y1_s, y2_s):
    msel = _msel()
    maskc = _maskcol()
    mg1, mg1t = _groupmats(128, 16)
    mg2, mg2t = _groupmats(256, 32)

    xv = x_ref[...].reshape(_MFULL, 64)

    ym1 = _dotd(xv[7:7 + _MV], w1m1_ref[0])
    y0 = _dotd(xv[8:8 + _MV], w10_ref[0])
    yp1 = _dotd(xv[9:9 + _MV], w1p1_ref[0])
    ye = ym1[:, 0:128] + y0[:, 0:128]
    yo = ym1[:, 128:256] + y0[:, 128:256] + yp1
    res_e = y0[:, 256:384]
    res_o = y0[:, 384:512]
    s1 = _dots(msel, ye) + _dots(msel, yo)
    s2 = _dots(msel, ye * ye) + _dots(msel, yo * yo)
    a, b = _ab_maps(s1, s2, mg1, mg1t, g1s_ref[0], g1b_ref[0], 4096.0)
    am = _expand(a)
    bm = _expand(b)
    y1_s[0:8, :] = jnp.zeros((8, 256), jnp.float32)
    y1_s[8 + _MV:_MFULL, :] = jnp.zeros((8, 256), jnp.float32)
    y1_s[8:8 + _MV, 0:128] = (_gelu(ye * am + bm) + res_e) * maskc
    y1_s[8:8 + _MV, 128:256] = (_gelu(yo * am + bm) + res_o) * maskc

    y0 = _dotd(y1_s[8:8 + _MV, :], w20_ref[0])
    yc = (_dotd(y1_s[7:7 + _MV, :], w2m1_ref[0])
          + y0[:, 0:256]
          + _dotd(y1_s[9:9 + _MV, 0:128], w2p1_ref[0]))
    s1 = _dots(msel, yc)
    s2 = _dots(msel, yc * yc)
    a, b = _ab_maps(s1, s2, mg2, mg2t, g2s_ref[0], g2b_ref[0], 4096.0)
    z2 = (_gelu(yc * _expand(a) + _expand(b)) + y0[:, 256:512]) * maskc
    y2_s[0:8, :] = jnp.zeros((8, 256), jnp.float32)
    y2_s[8 + _MV:_MFULL, :] = jnp.zeros((8, 256), jnp.float32)
    y2_s[8:8 + _MV, :] = z2

    y0 = _dotd(y2_s[8:8 + _MV, :], w30_ref[0])
    yc = (_dotd(y2_s[7:7 + _MV, :], w3m1_ref[0])
          + y0[:, 0:128]
          + _dotd(y2_s[9:9 + _MV, :], w3p1_ref[0]))
    s1 = _dots(msel, yc)
    s2 = _dots(msel, yc * yc)
    a, b = _ab_maps(s1, s2, mg1, mg1t, g3s_ref[0], g3b_ref[0], 2048.0)
    z3 = (_gelu(yc * _expand(a) + _expand(b)) + y0[:, 128:256]) * maskc

    s1 = _dots(msel, z3)
    s2 = _dots(msel, z3 * z3)
    a, b = _ab_maps(s1, s2, mg1, mg1t, ts_ref[0], tb_ref[0], 2048.0)
    o_ref[...] = a * (s1 / 128.0) + b


def _head1_body(he_ref, hi_ref, a_ref, b_ref, fb_ref, gw_ref, gb_ref,
                z_ref, gl_ref):
    t = _dotd(he_ref[...], a_ref[...]) + _dotd(hi_ref[...], b_ref[...]) \
        + fb_ref[...]
    z = _gelu(t)
    z_ref[...] = z
    gl_ref[...] = _dotd(z, gw_ref[...]) + gb_ref[...]


def _head2_body(z_ref, g_ref, w1_ref, b1_ref, lns_ref, lnb_ref, w2_ref, b2_ref,
                o_ref):
    z = z_ref[...]
    g = g_ref[...]
    acc = jnp.zeros((_B, _NC), jnp.float32)
    for e in range(_E):
        zz = _dotd(z, w1_ref[e]) + b1_ref[e]
        zz = _gelu(zz)
        mu = jnp.mean(zz, axis=1, keepdims=True)
        var = jnp.mean(zz * zz, axis=1, keepdims=True) - mu * mu
        zz = (zz - mu) * lax.rsqrt(var + _EPS) * lns_ref[e] + lnb_ref[e]
        le = _dotd(zz, w2_ref[e]) + b2_ref[e]
        acc = acc + g[:, e:e + 1] * le
    o_ref[...] = acc


def _sc_gates(gl_flat):
    mesh = plsc.VectorSubcoreMesh(core_axis_name="c", subcore_axis_name="s")
    chunk = _E * 16

    @functools.partial(
        pl.kernel, mesh=mesh,
        out_type=jax.ShapeDtypeStruct((_B * _E,), jnp.float32),
        scratch_types=[pltpu.VMEM((chunk,), jnp.float32),
                       pltpu.VMEM((chunk,), jnp.float32),
                       pltpu.SemaphoreType.DMA],
    )
    def k(gl_hbm, out_hbm, in_v, out_v, sem):
        wid = lax.axis_index("s") * 2 + lax.axis_index("c")

        @pl.when(wid < _B // 16)
        def _():
            base = wid * chunk
            pltpu.sync_copy(gl_hbm.at[pl.ds(base, chunk)], in_v)
            m1 = in_v[pl.ds(0, 16)]
            i1 = jnp.zeros((16,), jnp.int32)
            for e in range(1, _E):
                ve = in_v[pl.ds(e * 16, 16)]
                gt = ve > m1
                m1 = jnp.where(gt, ve, m1)
                i1 = jnp.where(gt, jnp.full((16,), e, jnp.int32), i1)
            m2 = jnp.full((16,), -3.0e38, jnp.float32)
            i2 = jnp.zeros((16,), jnp.int32)
            for e in range(_E):
                ve = in_v[pl.ds(e * 16, 16)]
                ok = jnp.logical_and(i1 != e, ve > m2)
                m2 = jnp.where(ok, ve, m2)
                i2 = jnp.where(ok, jnp.full((16,), e, jnp.int32), i2)
            t = jnp.exp(m2 - m1)
            denom = 1.0 + t
            p1 = 1.0 / denom
            p2 = t / denom
            zero = jnp.zeros((16,), jnp.float32)
            for e in range(_E):
                out_v[pl.ds(e * 16, 16)] = (jnp.where(i1 == e, p1, zero)
                                            + jnp.where(i2 == e, p2, zero))
            pltpu.sync_copy(out_v, out_hbm.at[pl.ds(base, chunk)])

    return k(gl_flat)


def _prep_encoder_weights(p):
    z16 = jnp.zeros((16, 128), jnp.float32)

    def taps1(pre):
        w = p[pre + '_w1']
        if w.shape[1] < 16:
            w = jnp.pad(w, ((0, 0), (0, 16 - w.shape[1]), (0, 0)))
        return [w[:, :, k].T for k in range(7)]

    def r1m(pre):
        r = p[pre + '_r1'][:, :, 0]
        if r.shape[1] < 16:
            r = jnp.pad(r, ((0, 0), (0, 16 - r.shape[1])))
        return r.T

    def w1m1(pre):
        t = taps1(pre)
        v_e = jnp.concatenate([z16, t[0], t[1], t[2]], axis=0)
        v_o = jnp.concatenate([z16, z16, z16, t[0]], axis=0)
        return jnp.concatenate([v_e, v_o], axis=1)

    def w10(pre):
        t = taps1(pre)
        r = r1m(pre)
        v_e = jnp.concatenate([t[3], t[4], t[5], t[6]], axis=0)
        v_o = jnp.concatenate([t[1], t[2], t[3], t[4]], axis=0)
        r_e = jnp.concatenate([r, z16, z16, z16], axis=0)
        r_o = jnp.concatenate([z16, z16, r, z16], axis=0)
        return jnp.concatenate([v_e, v_o, r_e, r_o], axis=1)

    def w1p1(pre):
        t = taps1(pre)
        return jnp.concatenate([t[5], t[6], z16, z16], axis=0)

    def tap(pre, name, d):
        return p[pre + name][:, :, d].T

    def w2m1(pre):
        return jnp.concatenate([tap(pre, '_w2', 0), tap(pre, '_w2', 1)], axis=0)

    def w20(pre):
        center = jnp.concatenate([tap(pre, '_w2', 2), tap(pre, '_w2', 3)], axis=0)
        resid = jnp.concatenate([p[pre + '_r2'][:, :, 0].T,
                                 jnp.zeros((128, 256), jnp.float32)], axis=0)
        return jnp.concatenate([center, resid], axis=1)

    def w30(pre):
        return jnp.concatenate([tap(pre, '_w3', 1), p[pre + '_r3'][:, :, 0].T],
                               axis=1)

    def stack(f):
        return jnp.stack([f('emg'), f('imu')], axis=0)

    def gn(name):
        return stack(lambda pre: p[pre + name][None, :])

    return {
        'w1m1': stack(w1m1), 'w10': stack(w10), 'w1p1': stack(w1p1),
        'g1s': gn('_g1s'), 'g1b': gn('_g1b'),
        'w2m1': stack(w2m1), 'w20': stack(w20),
        'w2p1': stack(lambda pre: tap(pre, '_w2', 4)),
        'g2s': gn('_g2s'), 'g2b': gn('_g2b'),
        'w3m1': stack(lambda pre: tap(pre, '_w3', 0)),
        'w30': stack(w30),
        'w3p1': stack(lambda pre: tap(pre, '_w3', 2)),
        'g3s': gn('_g3s'), 'g3b': gn('_g3b'),
        'gts': gn('_gts'), 'gtb': gn('_gtb'),
    }


def _forward(emg, imu, p, interpret=False, gates_fn=None):
    w = _prep_encoder_weights(p)

    emg_t = jnp.transpose(emg, (0, 2, 1))
    imu_t = jnp.pad(jnp.transpose(imu, (0, 2, 1)),
                    ((0, 0), (0, 0), (0, 10)))
    x = jnp.concatenate([emg_t, imu_t], axis=0)
    xq = jnp.pad(x.reshape(_SB, 128, 64), ((0, 0), (8, 8), (0, 0)))

    weights = [w['w1m1'], w['w10'], w['w1p1'], w['g1s'], w['g1b'],
               w['w2m1'], w['w20'], w['w2p1'], w['g2s'], w['g2b'],
               w['w3m1'], w['w30'], w['w3p1'], w['g3s'], w['g3b'],
               w['gts'], w['gtb']]
    in_specs = [pl.BlockSpec((_BB, _SREG, 64), lambda g: (g, 0, 0))]
    for wa in weights:
        in_specs.append(pl.BlockSpec((1,) + wa.shape[1:],
                                     lambda g: (g // (_G // 2), 0, 0)))

    h = pl.pallas_call(
        _enc_body,
        grid=(_G,),
        in_specs=in_specs,
        out_specs=pl.BlockSpec((_BB, _EMB), lambda g: (g, 0)),
        out_shape=jax.ShapeDtypeStruct((_SB, _EMB), jnp.float32),
        scratch_shapes=[pltpu.VMEM((_MFULL, 256), jnp.float32),
                        pltpu.VMEM((_MFULL, 256), jnp.float32)],
        interpret=interpret,
    )(xq, *weights)

    h_e = h[:_B]
    h_i = h[_B:]
    a = p['fuse_w'][:, :128].T
    b = p['fuse_w'][:, 128:].T

    z, gl = pl.pallas_call(
        _head1_body,
        out_shape=[jax.ShapeDtypeStruct((_B, _EMB), jnp.float32),
                   jax.ShapeDtypeStruct((_B, _E), jnp.float32)],
        interpret=interpret,
    )(h_e, h_i, a, b, p['fuse_b'][None, :], p['gate_w'].T, p['gate_b'][None, :])

    if gates_fn is None:
        gl_flat = jnp.transpose(gl.reshape(_B // 16, 16, _E),
                                (0, 2, 1)).reshape(_B * _E)
        gates = jnp.transpose(_sc_gates(gl_flat).reshape(_B // 16, _E, 16),
                              (0, 2, 1)).reshape(_B, _E)
    else:
        gates = gates_fn(gl)

    out = pl.pallas_call(
        _head2_body,
        out_shape=jax.ShapeDtypeStruct((_B, _NC), jnp.float32),
        interpret=interpret,
    )(z, gates, p['exp_w1'], p['exp_b1'][:, None, :],
      p['exp_lns'][:, None, :], p['exp_lnb'][:, None, :],
      p['exp_w2'], p['exp_b2'][:, None, :])
    return out


def kernel(emg, imu, params):
    return _forward(emg, imu, params)


# speedup vs baseline: 2.6745x; 1.3336x over previous
import functools

import jax
import jax.numpy as jnp
from jax import lax
from jax.experimental import pallas as pl
from jax.experimental.pallas import tpu as pltpu
from jax.experimental.pallas import tpu_sc as plsc

_B = 128
_SB = 2 * _B
_BB = 8
_G = _SB // _BB
_EMB = 128
_MID = 256
_E = 8
_NC = 10
_GROUPS = 8
_EPS = 1e-5
_SQRT2 = 1.4142135623730951
_PREC = lax.Precision.DEFAULT
_PREC_STAT = lax.Precision.HIGHEST

_SREG = 144
_MFULL = _BB * _SREG
_MV = _MFULL - 16


def _gelu(x):
    return x * 0.5 * (1.0 + lax.erf(x / _SQRT2))


def _dotd(a, b):
    return jnp.dot(a, b, preferred_element_type=jnp.float32, precision=_PREC)


def _dots(a, b):
    return jnp.dot(a, b, preferred_element_type=jnp.float32,
                   precision=_PREC_STAT)


def _rowsums(y):
    s1 = []
    s2 = []
    for s in range(_BB):
        blk = y[s * _SREG:s * _SREG + 128]
        s1.append(jnp.sum(blk, axis=0, keepdims=True))
        s2.append(jnp.sum(blk * blk, axis=0, keepdims=True))
    return jnp.concatenate(s1, axis=0), jnp.concatenate(s2, axis=0)


def _maskcol():
    r = lax.broadcasted_iota(jnp.int32, (_MV, 1), 0)
    return (r % _SREG < 128).astype(jnp.float32)


def _groupmats(c, cpg):
    gi = lax.broadcasted_iota(jnp.int32, (c, _GROUPS), 0) // cpg
    gj = lax.broadcasted_iota(jnp.int32, (c, _GROUPS), 1)
    mg = (gi == gj).astype(jnp.float32)
    ti = lax.broadcasted_iota(jnp.int32, (_GROUPS, c), 0)
    tj = lax.broadcasted_iota(jnp.int32, (_GROUPS, c), 1) // cpg
    mgt = (ti == tj).astype(jnp.float32)
    return mg, mgt


def _expand(a):
    c = a.shape[1]
    full = jnp.broadcast_to(a[:, None, :], (_BB, _SREG, c)).reshape(_MFULL, c)
    return full[0:_MV]


def _ab_maps(s1, s2, mg, mgt, gs, gb, n):
    g1 = _dots(s1, mg) / n
    g2 = _dots(s2, mg) / n
    grstd = lax.rsqrt(g2 - g1 * g1 + _EPS)
    mean_c = _dots(g1, mgt)
    rstd_c = _dots(grstd, mgt)
    a = rstd_c * gs
    b = gb - mean_c * a
    return a, b


def _enc_body(x_ref, w1m1_ref, w10_ref, w1p1_ref, g1s_ref, g1b_ref,
              w2m1_ref, w20_ref, w2p1_ref, g2s_ref, g2b_ref,
              w3m1_ref, w30_ref, w3p1_ref, g3s_ref, g3b_ref,
              ts_ref, tb_ref, o_ref, y1_s, y2_s):
    maskc = _maskcol()
    mg1, mg1t = _groupmats(128, 16)
    mg2, mg2t = _groupmats(256, 32)

    xv = x_ref[...].reshape(_MFULL, 64)

    ym1 = _dotd(xv[7:7 + _MV], w1m1_ref[0])
    y0 = _dotd(xv[8:8 + _MV], w10_ref[0])
    yp1 = _dotd(xv[9:9 + _MV], w1p1_ref[0])
    ye = ym1[:, 0:128] + y0[:, 0:128]
    yo = ym1[:, 128:256] + y0[:, 128:256] + yp1
    res_e = y0[:, 256:384]
    res_o = y0[:, 384:512]
    s1e, s2e = _rowsums(ye)
    s1o, s2o = _rowsums(yo)
    s1 = s1e + s1o
    s2 = s2e + s2o
    a, b = _ab_maps(s1, s2, mg1, mg1t, g1s_ref[0], g1b_ref[0], 4096.0)
    am = _expand(a)
    bm = _expand(b)
    y1_s[0:8, :] = jnp.zeros((8, 256), jnp.float32)
    y1_s[8 + _MV:_MFULL, :] = jnp.zeros((8, 256), jnp.float32)
    y1_s[8:8 + _MV, 0:128] = (_gelu(ye * am + bm) + res_e) * maskc
    y1_s[8:8 + _MV, 128:256] = (_gelu(yo * am + bm) + res_o) * maskc

    y0 = _dotd(y1_s[8:8 + _MV, :], w20_ref[0])
    yc = (_dotd(y1_s[7:7 + _MV, :], w2m1_ref[0])
          + y0[:, 0:256]
          + _dotd(y1_s[9:9 + _MV, 0:128], w2p1_ref[0]))
    s1, s2 = _rowsums(yc)
    a, b = _ab_maps(s1, s2, mg2, mg2t, g2s_ref[0], g2b_ref[0], 4096.0)
    z2 = (_gelu(yc * _expand(a) + _expand(b)) + y0[:, 256:512]) * maskc
    y2_s[0:8, :] = jnp.zeros((8, 256), jnp.float32)
    y2_s[8 + _MV:_MFULL, :] = jnp.zeros((8, 256), jnp.float32)
    y2_s[8:8 + _MV, :] = z2

    y0 = _dotd(y2_s[8:8 + _MV, :], w30_ref[0])
    yc = (_dotd(y2_s[7:7 + _MV, :], w3m1_ref[0])
          + y0[:, 0:128]
          + _dotd(y2_s[9:9 + _MV, :], w3p1_ref[0]))
    s1, s2 = _rowsums(yc)
    a, b = _ab_maps(s1, s2, mg1, mg1t, g3s_ref[0], g3b_ref[0], 2048.0)
    z3 = (_gelu(yc * _expand(a) + _expand(b)) + y0[:, 128:256]) * maskc

    s1, s2 = _rowsums(z3)
    a, b = _ab_maps(s1, s2, mg1, mg1t, ts_ref[0], tb_ref[0], 2048.0)
    o_ref[...] = a * (s1 / 128.0) + b


def _head1_body(he_ref, hi_ref, a_ref, b_ref, fb_ref, gw_ref, gb_ref,
                z_ref, gl_ref):
    t = _dotd(he_ref[...], a_ref[...]) + _dotd(hi_ref[...], b_ref[...]) \
        + fb_ref[...]
    z = _gelu(t)
    z_ref[...] = z
    gl_ref[...] = _dotd(z, gw_ref[...]) + gb_ref[...]


def _head2_body(z_ref, g_ref, w1_ref, b1_ref, lns_ref, lnb_ref, w2_ref, b2_ref,
                o_ref):
    z = z_ref[...]
    g = g_ref[...]
    acc = jnp.zeros((_B, _NC), jnp.float32)
    for e in range(_E):
        zz = _dotd(z, w1_ref[e]) + b1_ref[e]
        zz = _gelu(zz)
        mu = jnp.mean(zz, axis=1, keepdims=True)
        var = jnp.mean(zz * zz, axis=1, keepdims=True) - mu * mu
        zz = (zz - mu) * lax.rsqrt(var + _EPS) * lns_ref[e] + lnb_ref[e]
        le = _dotd(zz, w2_ref[e]) + b2_ref[e]
        acc = acc + g[:, e:e + 1] * le
    o_ref[...] = acc


def _sc_gates(gl_flat):
    mesh = plsc.VectorSubcoreMesh(core_axis_name="c", subcore_axis_name="s")
    chunk = _E * 16

    @functools.partial(
        pl.kernel, mesh=mesh,
        out_type=jax.ShapeDtypeStruct((_B * _E,), jnp.float32),
        scratch_types=[pltpu.VMEM((chunk,), jnp.float32),
                       pltpu.VMEM((chunk,), jnp.float32),
                       pltpu.SemaphoreType.DMA],
    )
    def k(gl_hbm, out_hbm, in_v, out_v, sem):
        wid = lax.axis_index("s") * 2 + lax.axis_index("c")

        @pl.when(wid < _B // 16)
        def _():
            base = wid * chunk
            pltpu.sync_copy(gl_hbm.at[pl.ds(base, chunk)], in_v)
            m1 = in_v[pl.ds(0, 16)]
            i1 = jnp.zeros((16,), jnp.int32)
            for e in range(1, _E):
                ve = in_v[pl.ds(e * 16, 16)]
                gt = ve > m1
                m1 = jnp.where(gt, ve, m1)
                i1 = jnp.where(gt, jnp.full((16,), e, jnp.int32), i1)
            m2 = jnp.full((16,), -3.0e38, jnp.float32)
            i2 = jnp.zeros((16,), jnp.int32)
            for e in range(_E):
                ve = in_v[pl.ds(e * 16, 16)]
                ok = jnp.logical_and(i1 != e, ve > m2)
                m2 = jnp.where(ok, ve, m2)
                i2 = jnp.where(ok, jnp.full((16,), e, jnp.int32), i2)
            t = jnp.exp(m2 - m1)
            denom = 1.0 + t
            p1 = 1.0 / denom
            p2 = t / denom
            zero = jnp.zeros((16,), jnp.float32)
            for e in range(_E):
                out_v[pl.ds(e * 16, 16)] = (jnp.where(i1 == e, p1, zero)
                                            + jnp.where(i2 == e, p2, zero))
            pltpu.sync_copy(out_v, out_hbm.at[pl.ds(base, chunk)])

    return k(gl_flat)


def _prep_encoder_weights(p):
    z16 = jnp.zeros((16, 128), jnp.float32)

    def taps1(pre):
        w = p[pre + '_w1']
        if w.shape[1] < 16:
            w = jnp.pad(w, ((0, 0), (0, 16 - w.shape[1]), (0, 0)))
        return [w[:, :, k].T for k in range(7)]

    def r1m(pre):
        r = p[pre + '_r1'][:, :, 0]
        if r.shape[1] < 16:
            r = jnp.pad(r, ((0, 0), (0, 16 - r.shape[1])))
        return r.T

    def w1m1(pre):
        t = taps1(pre)
        v_e = jnp.concatenate([z16, t[0], t[1], t[2]], axis=0)
        v_o = jnp.concatenate([z16, z16, z16, t[0]], axis=0)
        return jnp.concatenate([v_e, v_o], axis=1)

    def w10(pre):
        t = taps1(pre)
        r = r1m(pre)
        v_e = jnp.concatenate([t[3], t[4], t[5], t[6]], axis=0)
        v_o = jnp.concatenate([t[1], t[2], t[3], t[4]], axis=0)
        r_e = jnp.concatenate([r, z16, z16, z16], axis=0)
        r_o = jnp.concatenate([z16, z16, r, z16], axis=0)
        return jnp.concatenate([v_e, v_o, r_e, r_o], axis=1)

    def w1p1(pre):
        t = taps1(pre)
        return jnp.concatenate([t[5], t[6], z16, z16], axis=0)

    def tap(pre, name, d):
        return p[pre + name][:, :, d].T

    def w2m1(pre):
        return jnp.concatenate([tap(pre, '_w2', 0), tap(pre, '_w2', 1)], axis=0)

    def w20(pre):
        center = jnp.concatenate([tap(pre, '_w2', 2), tap(pre, '_w2', 3)], axis=0)
        resid = jnp.concatenate([p[pre + '_r2'][:, :, 0].T,
                                 jnp.zeros((128, 256), jnp.float32)], axis=0)
        return jnp.concatenate([center, resid], axis=1)

    def w30(pre):
        return jnp.concatenate([tap(pre, '_w3', 1), p[pre + '_r3'][:, :, 0].T],
                               axis=1)

    def stack(f):
        return jnp.stack([f('emg'), f('imu')], axis=0)

    def gn(name):
        return stack(lambda pre: p[pre + name][None, :])

    return {
        'w1m1': stack(w1m1), 'w10': stack(w10), 'w1p1': stack(w1p1),
        'g1s': gn('_g1s'), 'g1b': gn('_g1b'),
        'w2m1': stack(w2m1), 'w20': stack(w20),
        'w2p1': stack(lambda pre: tap(pre, '_w2', 4)),
        'g2s': gn('_g2s'), 'g2b': gn('_g2b'),
        'w3m1': stack(lambda pre: tap(pre, '_w3', 0)),
        'w30': stack(w30),
        'w3p1': stack(lambda pre: tap(pre, '_w3', 2)),
        'g3s': gn('_g3s'), 'g3b': gn('_g3b'),
        'gts': gn('_gts'), 'gtb': gn('_gtb'),
    }


def _forward(emg, imu, p, interpret=False, gates_fn=None):
    w = _prep_encoder_weights(p)

    emg_t = jnp.transpose(emg, (0, 2, 1))
    imu_t = jnp.pad(jnp.transpose(imu, (0, 2, 1)),
                    ((0, 0), (0, 0), (0, 10)))
    x = jnp.concatenate([emg_t, imu_t], axis=0)
    xq = jnp.pad(x.reshape(_SB, 128, 64), ((0, 0), (8, 8), (0, 0)))

    weights = [w['w1m1'], w['w10'], w['w1p1'], w['g1s'], w['g1b'],
               w['w2m1'], w['w20'], w['w2p1'], w['g2s'], w['g2b'],
               w['w3m1'], w['w30'], w['w3p1'], w['g3s'], w['g3b'],
               w['gts'], w['gtb']]
    in_specs = [pl.BlockSpec((_BB, _SREG, 64), lambda g: (g, 0, 0))]
    for wa in weights:
        in_specs.append(pl.BlockSpec((1,) + wa.shape[1:],
                                     lambda g: (g // (_G // 2), 0, 0)))

    h = pl.pallas_call(
        _enc_body,
        grid=(_G,),
        in_specs=in_specs,
        out_specs=pl.BlockSpec((_BB, _EMB), lambda g: (g, 0)),
        out_shape=jax.ShapeDtypeStruct((_SB, _EMB), jnp.float32),
        scratch_shapes=[pltpu.VMEM((_MFULL, 256), jnp.float32),
                        pltpu.VMEM((_MFULL, 256), jnp.float32)],
        interpret=interpret,
    )(xq, *weights)

    h_e = h[:_B]
    h_i = h[_B:]
    a = p['fuse_w'][:, :128].T
    b = p['fuse_w'][:, 128:].T

    z, gl = pl.pallas_call(
        _head1_body,
        out_shape=[jax.ShapeDtypeStruct((_B, _EMB), jnp.float32),
                   jax.ShapeDtypeStruct((_B, _E), jnp.float32)],
        interpret=interpret,
    )(h_e, h_i, a, b, p['fuse_b'][None, :], p['gate_w'].T, p['gate_b'][None, :])

    if gates_fn is None:
        gl_flat = jnp.transpose(gl.reshape(_B // 16, 16, _E),
                                (0, 2, 1)).reshape(_B * _E)
        gates = jnp.transpose(_sc_gates(gl_flat).reshape(_B // 16, _E, 16),
                              (0, 2, 1)).reshape(_B, _E)
    else:
        gates = gates_fn(gl)

    out = pl.pallas_call(
        _head2_body,
        out_shape=jax.ShapeDtypeStruct((_B, _NC), jnp.float32),
        interpret=interpret,
    )(z, gates, p['exp_w1'], p['exp_b1'][:, None, :],
      p['exp_lns'][:, None, :], p['exp_lnb'][:, None, :],
      p['exp_w2'], p['exp_b2'][:, None, :])
    return out


def kernel(emg, imu, params):
    return _forward(emg, imu, params)
